# baseline (speedup 1.0000x reference)
import jax
import jax.numpy as jnp
from jax import lax
from jax.experimental import pallas as pl
from jax.experimental.pallas import tpu as pltpu
from jax.experimental.pallas import tpu_sc as plsc

_B, _S, _F = 8, 1024, 256
_NC, _NS, _L = 2, 16, 16
_NW = _NC * _NS
_RPW = _B * _S // _NW
_CR = 128
_NCH = _RPW // _CR

_vector_mesh = plsc.VectorSubcoreMesh(core_axis_name="c", subcore_axis_name="s")


def _chunk_min(buf, accs):
    def row_body(r, accs):
        new = list(accs)
        for j in range(_F // _L):
            new[j % 8] = jnp.minimum(new[j % 8], buf[r, pl.ds(j * _L, _L)])
        return tuple(new)

    return lax.fori_loop(0, _CR, row_body, accs)


def _sc_min_body(x_hbm, pmin_hbm, buf0, buf1, acc, sem0, sem1, osem):
    wid = lax.axis_index("s") * _NC + lax.axis_index("c")
    b = wid // 4
    r0 = (wid % 4) * _RPW
    bufs = [buf0, buf1]
    sems = [sem0, sem1]

    pltpu.make_async_copy(x_hbm.at[b, pl.ds(r0, _CR)], buf0, sem0).start()
    accs = tuple(jnp.full((_L,), jnp.inf, jnp.float32) for _ in range(8))
    for k in range(_NCH):
        buf, sem = bufs[k % 2], sems[k % 2]
        pltpu.make_async_copy(
            x_hbm.at[b, pl.ds(r0 + k * _CR, _CR)], buf, sem
        ).wait()
        if k + 1 < _NCH:
            nbuf, nsem = bufs[(k + 1) % 2], sems[(k + 1) % 2]
            pltpu.make_async_copy(
                x_hbm.at[b, pl.ds(r0 + (k + 1) * _CR, _CR)], nbuf, nsem
            ).start()
        accs = _chunk_min(buf, accs)

    m = accs[0]
    for a in accs[1:]:
        m = jnp.minimum(m, a)
    acc[...] = m
    cp = pltpu.make_async_copy(acc, pmin_hbm.at[wid], osem)
    cp.start()
    cp.wait()


def _sc_partial_min(x):
    k = pl.kernel(
        _sc_min_body,
        out_type=jax.ShapeDtypeStruct((_NW, _L), jnp.float32),
        mesh=_vector_mesh,
        scratch_types=[
            pltpu.VMEM((_CR, _F), jnp.float32),
            pltpu.VMEM((_CR, _F), jnp.float32),
            pltpu.VMEM((_L,), jnp.float32),
            pltpu.SemaphoreType.DMA,
            pltpu.SemaphoreType.DMA,
            pltpu.SemaphoreType.DMA,
        ],
        compiler_params=pltpu.CompilerParams(use_tc_tiling_on_sc=True),
    )
    return k(x)


def _tc_copy_fill_body(pmin_vmem, in_hbm, out_hbm, stage, fillbuf, in_sems,
                       out_sems, fill_sems):
    for c in range(2):
        pltpu.make_async_copy(
            in_hbm.at[pl.ds(c * 4, 4)], stage.at[c], in_sems.at[c]
        ).start()
    minv = jnp.min(pmin_vmem[...]) - 1.0
    fillbuf[...] = jnp.full((_S, _F), minv, jnp.float32)
    for b in range(_B):
        pltpu.make_async_copy(
            fillbuf, out_hbm.at[b, _S : 2 * _S], fill_sems.at[b]
        ).start()
    for c in range(2):
        pltpu.make_async_copy(
            in_hbm.at[pl.ds(c * 4, 4)], stage.at[c], in_sems.at[c]
        ).wait()
        pltpu.make_async_copy(
            stage.at[c], out_hbm.at[pl.ds(c * 4, 4), 0:_S], out_sems.at[c]
        ).start()
    for c in range(2):
        pltpu.make_async_copy(
            stage.at[c], out_hbm.at[pl.ds(c * 4, 4), 0:_S], out_sems.at[c]
        ).wait()
    for b in range(_B):
        pltpu.make_async_copy(
            fillbuf, out_hbm.at[b, _S : 2 * _S], fill_sems.at[b]
        ).wait()


def _tc_copy_fill(pmin, inputs):
    return pl.pallas_call(
        _tc_copy_fill_body,
        in_specs=[
            pl.BlockSpec(memory_space=pltpu.MemorySpace.VMEM),
            pl.BlockSpec(memory_space=pltpu.MemorySpace.HBM),
        ],
        out_specs=pl.BlockSpec(memory_space=pltpu.MemorySpace.HBM),
        out_shape=jax.ShapeDtypeStruct((_B, 2 * _S, _F), inputs.dtype),
        scratch_shapes=[
            pltpu.VMEM((2, 4, _S, _F), jnp.float32),
            pltpu.VMEM((_S, _F), jnp.float32),
            pltpu.SemaphoreType.DMA((2,)),
            pltpu.SemaphoreType.DMA((2,)),
            pltpu.SemaphoreType.DMA((_B,)),
        ],
    )(pmin, inputs)


def kernel(inputs):
    pmin = _sc_partial_min(inputs)
    return _tc_copy_fill(pmin, inputs)


# speedup vs baseline: 3.9874x; 3.9874x over previous
import jax
import jax.numpy as jnp
from jax.experimental import pallas as pl
from jax.experimental.pallas import tpu as pltpu

_B, _S, _F = 8, 1024, 256
_BPC = 4
_NCH = _B // _BPC


def _body(in_hbm, out_hbm, stage, fillbuf, in_sems, out_sems, fill_sems):
    for c in range(_NCH):
        b = c * _BPC
        pltpu.make_async_copy(
            in_hbm.at[pl.ds(b, _BPC)], stage.at[c], in_sems.at[c]
        ).start()

    minval = None
    for c in range(_NCH):
        b = c * _BPC
        pltpu.make_async_copy(
            in_hbm.at[pl.ds(b, _BPC)], stage.at[c], in_sems.at[c]
        ).wait()
        pltpu.make_async_copy(
            stage.at[c], out_hbm.at[pl.ds(b, _BPC), 0:_S], out_sems.at[c]
        ).start()
        cmin = jnp.min(stage[c])
        minval = cmin if minval is None else jnp.minimum(minval, cmin)

    fillbuf[...] = jnp.full(fillbuf.shape, minval - 1.0, fillbuf.dtype)
    for b in range(_B):
        pltpu.make_async_copy(
            fillbuf, out_hbm.at[b, _S : 2 * _S], fill_sems.at[b]
        ).start()

    for c in range(_NCH):
        b = c * _BPC
        pltpu.make_async_copy(
            stage.at[c], out_hbm.at[pl.ds(b, _BPC), 0:_S], out_sems.at[c]
        ).wait()
    for b in range(_B):
        pltpu.make_async_copy(
            fillbuf, out_hbm.at[b, _S : 2 * _S], fill_sems.at[b]
        ).wait()


def kernel(inputs):
    return pl.pallas_call(
        _body,
        in_specs=[pl.BlockSpec(memory_space=pltpu.MemorySpace.HBM)],
        out_specs=pl.BlockSpec(memory_space=pltpu.MemorySpace.HBM),
        out_shape=jax.ShapeDtypeStruct((_B, 2 * _S, _F), inputs.dtype),
        scratch_shapes=[
            pltpu.VMEM((_NCH, _BPC, _S, _F), jnp.float32),
            pltpu.VMEM((_S, _F), jnp.float32),
            pltpu.SemaphoreType.DMA((_NCH,)),
            pltpu.SemaphoreType.DMA((_NCH,)),
            pltpu.SemaphoreType.DMA((_B,)),
        ],
    )(inputs)
